# Initial kernel scaffold; baseline (speedup 1.0000x reference)
import jax
import jax.numpy as jnp
from jax.experimental import pallas as pl


def kernel(x, pos, edge_attr, edge_index, W_emb, b_emb, ew1, eb1, ew2, eb2, ew3, eb3, cw1, cb1, cw2, cb2, cw3, cb3, nw1, nb1, nw2, nb2, nw3, nb3):
    raise NotImplementedError("write your pallas kernel here")


# speedup vs baseline: 4.1261x; 4.1261x over previous
import functools

import numpy as np
import jax
import jax.numpy as jnp
from jax import lax
from jax.experimental import pallas as pl
from jax.experimental.pallas import tpu as pltpu
from jax.experimental.pallas import tpu_sc as plsc

N = 10000
NPAD = 10240
E = 320000
F = 128
H = 64

TW = 128
AW = 32

NTILES = 32
EPT = E // NTILES
GC = 400
ROWS_PER_TILE = NPAD // 16

BN = 1024
BE = 2000

_F32 = jnp.float32


def _leaky(v):
    return jnp.where(v >= 0, v, 0.3 * v)


def _dot(a, b):
    return jnp.dot(a, b, preferred_element_type=_F32)


def _init_body(x_ref, pos_ref, wemb_ref, bemb_ref, ew1_ref,
               h_ref, t1_ref, t2_ref):
    h0 = _dot(x_ref[...], wemb_ref[...]) + bemb_ref[...]
    ew1 = ew1_ref[...]
    u = _dot(h0, ew1[1:65, :])
    v = _dot(h0, ew1[66:130, :])
    p = pos_ref[...]
    z = jnp.zeros((BN, TW - H - 3), _F32)
    h_ref[...] = h0
    t1_ref[...] = jnp.concatenate([u, p, z], axis=1)
    t2_ref[...] = jnp.concatenate([v, -p, z], axis=1)


_init_call = pl.pallas_call(
    _init_body,
    grid=(NPAD // BN,),
    in_specs=[
        pl.BlockSpec((BN, F), lambda i: (i, 0)),
        pl.BlockSpec((BN, 3), lambda i: (i, 0)),
        pl.BlockSpec((F, H), lambda i: (0, 0)),
        pl.BlockSpec((1, H), lambda i: (0, 0)),
        pl.BlockSpec((2 * H + 7, H), lambda i: (0, 0)),
    ],
    out_specs=[
        pl.BlockSpec((BN, H), lambda i: (i, 0)),
        pl.BlockSpec((BN, TW), lambda i: (i, 0)),
        pl.BlockSpec((BN, TW), lambda i: (i, 0)),
    ],
    out_shape=[
        jax.ShapeDtypeStruct((NPAD, H), _F32),
        jax.ShapeDtypeStruct((NPAD, TW), _F32),
        jax.ShapeDtypeStruct((NPAD, TW), _F32),
    ],
)


def _edge_body(g1_ref, g2_ref, ea_ref, ew1t_ref, eb1_ref, ew2_ref, eb2_ref,
               ew3_ref, eb3_ref, cw1_ref, cb1_ref, cw2_ref, cb2_ref,
               cw3_ref, cb3_ref, out_ref):
    g1 = g1_ref[...]
    g2 = g2_ref[...]
    diff = g1[:, H:H + 3] + g2[:, H:H + 3]
    radial = jnp.sum(diff * diff, axis=1, keepdims=True)
    ew1t = ew1t_ref[...]
    pre = (g1[:, :H] + g2[:, :H]
           + radial * ew1t[0:1, :]
           + _dot(ea_ref[...], ew1t[1:5, :])
           + eb1_ref[...])
    h1 = _leaky(pre)
    h2 = _leaky(_dot(h1, ew2_ref[...]) + eb2_ref[...])
    ef = _dot(h2, ew3_ref[...]) + eb3_ref[...]
    c1 = _leaky(_dot(ef, cw1_ref[...]) + cb1_ref[...])
    c2 = _leaky(_dot(c1, cw2_ref[...]) + cb2_ref[...])
    cm = _dot(c2, cw3_ref[...]) + cb3_ref[...]
    trans = jnp.clip(diff * cm, -100.0, 100.0)
    ones = jnp.ones((BE, 1), _F32)
    zpad = jnp.zeros((BE, AW - 20), _F32)
    out_ref[...] = jnp.concatenate([ef, trans, ones, zpad], axis=1)


_edge_call = pl.pallas_call(
    _edge_body,
    grid=(E // BE,),
    in_specs=[
        pl.BlockSpec((BE, TW), lambda i: (i, 0)),
        pl.BlockSpec((BE, TW), lambda i: (i, 0)),
        pl.BlockSpec((BE, 4), lambda i: (i, 0)),
        pl.BlockSpec((5, H), lambda i: (0, 0)),
        pl.BlockSpec((1, H), lambda i: (0, 0)),
        pl.BlockSpec((H, H // 2), lambda i: (0, 0)),
        pl.BlockSpec((1, H // 2), lambda i: (0, 0)),
        pl.BlockSpec((H // 2, H // 4), lambda i: (0, 0)),
        pl.BlockSpec((1, H // 4), lambda i: (0, 0)),
        pl.BlockSpec((H // 4, H // 2), lambda i: (0, 0)),
        pl.BlockSpec((1, H // 2), lambda i: (0, 0)),
        pl.BlockSpec((H // 2, H // 2), lambda i: (0, 0)),
        pl.BlockSpec((1, H // 2), lambda i: (0, 0)),
        pl.BlockSpec((H // 2, 3), lambda i: (0, 0)),
        pl.BlockSpec((1, 3), lambda i: (0, 0)),
    ],
    out_specs=pl.BlockSpec((BE, AW), lambda i: (i, 0)),
    out_shape=jax.ShapeDtypeStruct((E, AW), _F32),
)


def _make_node_body(t_cur, t_next, dt):
    def body(h_ref, p_ref, aggA_ref, aggB_ref, nw1_ref, nb1_ref, nw2_ref,
             nb2_ref, nw3_ref, nb3_ref, ew1_ref,
             hn_ref, pn_ref, t1_ref, t2_ref):
        h = h_ref[...]
        p = p_ref[...]
        agg = aggA_ref[...] + aggB_ref[...]
        ef = agg[:, 0:16]
        sums = agg[:, 16:19]
        deg = agg[:, 19:20]
        nw1 = nw1_ref[...]
        o1 = _leaky(t_cur * nw1[0:1, :]
                    + _dot(h, nw1[1:H + 1, :])
                    + _dot(ef, nw1[H + 1:H + 17, :])
                    + nb1_ref[...])
        o2 = _leaky(_dot(o1, nw2_ref[...]) + nb2_ref[...])
        mlp = _dot(o2, nw3_ref[...]) + nb3_ref[...]
        hn = h + dt * mlp
        coord = p + sums / jnp.maximum(deg, 1.0)
        pn = p + dt * coord
        ew1 = ew1_ref[...]
        u = t_next * ew1[0:1, :] + _dot(hn, ew1[1:65, :])
        v = t_next * ew1[65:66, :] + _dot(hn, ew1[66:130, :])
        z = jnp.zeros((BN, TW - H - 3), _F32)
        hn_ref[...] = hn
        pn_ref[...] = pn
        t1_ref[...] = jnp.concatenate([u, pn, z], axis=1)
        t2_ref[...] = jnp.concatenate([v, -pn, z], axis=1)
    return body


def _make_node_call(t_cur, t_next, dt):
    return pl.pallas_call(
        _make_node_body(t_cur, t_next, dt),
        grid=(NPAD // BN,),
        in_specs=[
            pl.BlockSpec((BN, H), lambda i: (i, 0)),
            pl.BlockSpec((BN, 3), lambda i: (i, 0)),
            pl.BlockSpec((BN, AW), lambda i: (i, 0)),
            pl.BlockSpec((BN, AW), lambda i: (i + NPAD // BN, 0)),
            pl.BlockSpec((H + H // 4 + 1, 2 * H), lambda i: (0, 0)),
            pl.BlockSpec((1, 2 * H), lambda i: (0, 0)),
            pl.BlockSpec((2 * H, 2 * H), lambda i: (0, 0)),
            pl.BlockSpec((1, 2 * H), lambda i: (0, 0)),
            pl.BlockSpec((2 * H, H), lambda i: (0, 0)),
            pl.BlockSpec((1, H), lambda i: (0, 0)),
            pl.BlockSpec((2 * H + 7, H), lambda i: (0, 0)),
        ],
        out_specs=[
            pl.BlockSpec((BN, H), lambda i: (i, 0)),
            pl.BlockSpec((BN, 3), lambda i: (i, 0)),
            pl.BlockSpec((BN, TW), lambda i: (i, 0)),
            pl.BlockSpec((BN, TW), lambda i: (i, 0)),
        ],
        out_shape=[
            jax.ShapeDtypeStruct((NPAD, H), _F32),
            jax.ShapeDtypeStruct((NPAD, 3), _F32),
            jax.ShapeDtypeStruct((NPAD, TW), _F32),
            jax.ShapeDtypeStruct((NPAD, TW), _F32),
        ],
    )


_TS = np.linspace(0.0, 2.0, 4).astype(np.float32)
_node_calls = [
    _make_node_call(float(_TS[i]), float(_TS[i + 1]), float(_TS[i + 1] - _TS[i]))
    for i in range(3)
]


_MESH = plsc.VectorSubcoreMesh(core_axis_name="c", subcore_axis_name="s")


@functools.partial(
    pl.kernel,
    out_type=[
        jax.ShapeDtypeStruct((E, TW), _F32),
        jax.ShapeDtypeStruct((E, TW), _F32),
    ],
    mesh=_MESH,
    scratch_types=[
        pltpu.VMEM((GC,), jnp.int32),
        pltpu.VMEM((GC, TW), _F32),
        pltpu.SemaphoreType.DMA,
    ],
)
def _gather_call(t1_hbm, t2_hbm, row_hbm, col_hbm, g1_out, g2_out,
                 idx_v, buf, sem):
    wid = lax.axis_index("s") * 2 + lax.axis_index("c")
    base = wid * EPT

    def body(i, carry):
        off = base + i * GC
        pltpu.sync_copy(row_hbm.at[pl.ds(off, GC)], idx_v)
        pltpu.async_copy(t1_hbm.at[idx_v], buf, sem).wait()
        pltpu.sync_copy(buf, g1_out.at[pl.ds(off, GC)])
        pltpu.sync_copy(col_hbm.at[pl.ds(off, GC)], idx_v)
        pltpu.async_copy(t2_hbm.at[idx_v], buf, sem).wait()
        pltpu.sync_copy(buf, g2_out.at[pl.ds(off, GC)])
        return carry

    lax.fori_loop(0, EPT // GC, body, 0)


@functools.partial(
    pl.kernel,
    out_type=jax.ShapeDtypeStruct((2 * NPAD, AW), _F32),
    mesh=_MESH,
    scratch_types=[
        pltpu.VMEM((GC,), jnp.int32),
        pltpu.VMEM((GC, AW), _F32),
        pltpu.VMEM((ROWS_PER_TILE, AW), _F32),
        pltpu.VMEM_SHARED((NPAD, AW), _F32),
    ],
    compiler_params=pltpu.CompilerParams(use_tc_tiling_on_sc=False),
)
def _scatter_call(efo_hbm, row_hbm, zeros_hbm, out_hbm, idx_v, buf, zbuf, acc):
    c = lax.axis_index("c")
    s = lax.axis_index("s")
    wid = s * 2 + c
    myrows = s * ROWS_PER_TILE
    pltpu.sync_copy(zeros_hbm.at[pl.ds(myrows, ROWS_PER_TILE)], zbuf)
    pltpu.sync_copy(zbuf, acc.at[pl.ds(myrows, ROWS_PER_TILE)])
    plsc.subcore_barrier()
    base = wid * EPT

    def body(i, carry):
        off = base + i * GC
        pltpu.sync_copy(row_hbm.at[pl.ds(off, GC)], idx_v)
        pltpu.sync_copy(efo_hbm.at[pl.ds(off, GC)], buf)
        pltpu.sync_copy(buf, acc.at[idx_v], add=True)
        return carry

    lax.fori_loop(0, EPT // GC, body, 0)
    plsc.subcore_barrier()
    pltpu.sync_copy(acc.at[pl.ds(myrows, ROWS_PER_TILE)], zbuf)
    pltpu.sync_copy(zbuf, out_hbm.at[pl.ds(c * NPAD + myrows, ROWS_PER_TILE)])


def kernel(x, pos, edge_attr, edge_index, W_emb, b_emb, ew1, eb1, ew2, eb2,
           ew3, eb3, cw1, cb1, cw2, cb2, cw3, cb3, nw1, nb1, nw2, nb2,
           nw3, nb3):
    row = edge_index[0]
    col = edge_index[1]
    xp = jnp.pad(x, ((0, NPAD - N), (0, 0)))
    pp = jnp.pad(pos, ((0, NPAD - N), (0, 0)))

    b_emb2 = b_emb.reshape(1, -1)
    eb1_2 = eb1.reshape(1, -1)
    eb2_2 = eb2.reshape(1, -1)
    eb3_2 = eb3.reshape(1, -1)
    cb1_2 = cb1.reshape(1, -1)
    cb2_2 = cb2.reshape(1, -1)
    cb3_2 = cb3.reshape(1, -1)
    nb1_2 = nb1.reshape(1, -1)
    nb2_2 = nb2.reshape(1, -1)
    nb3_2 = nb3.reshape(1, -1)
    ew1t = ew1[130:135]
    zeros_acc = jnp.zeros((NPAD, AW), _F32)

    h, t1, t2 = _init_call(xp, pp, W_emb, b_emb2, ew1)
    p = pp
    traj = [h]
    for i in range(3):
        g1, g2 = _gather_call(t1, t2, row, col)
        efo = _edge_call(g1, g2, edge_attr, ew1t, eb1_2, ew2, eb2_2, ew3,
                         eb3_2, cw1, cb1_2, cw2, cb2_2, cw3, cb3_2)
        aggs = _scatter_call(efo, row, zeros_acc)
        h, p, t1, t2 = _node_calls[i](h, p, aggs, aggs, nw1, nb1_2, nw2,
                                      nb2_2, nw3, nb3_2, ew1)
        traj.append(h)
    return jnp.stack([hh[:N] for hh in traj], axis=0)


# speedup vs baseline: 4.3719x; 1.0596x over previous
import functools

import numpy as np
import jax
import jax.numpy as jnp
from jax import lax
from jax.experimental import pallas as pl
from jax.experimental.pallas import tpu as pltpu
from jax.experimental.pallas import tpu_sc as plsc

N = 10000
NPAD = 10240
E = 320000
F = 128
H = 64

TW = 128
AW = 32

NTILES = 32
EPT = E // NTILES
GC = 400
ROWS_PER_TILE = NPAD // 16

BN = 1024
BE = 2000

_F32 = jnp.float32


def _leaky(v):
    return jnp.where(v >= 0, v, 0.3 * v)


def _dot(a, b):
    return jnp.dot(a, b, preferred_element_type=_F32)


def _init_body(x_ref, pos_ref, wemb_ref, bemb_ref, ew1_ref,
               h_ref, t1_ref, t2_ref):
    h0 = _dot(x_ref[...], wemb_ref[...]) + bemb_ref[...]
    ew1 = ew1_ref[...]
    u = _dot(h0, ew1[1:65, :])
    v = _dot(h0, ew1[66:130, :])
    p = pos_ref[...]
    z = jnp.zeros((BN, TW - H - 3), _F32)
    h_ref[...] = h0
    t1_ref[...] = jnp.concatenate([u, p, z], axis=1)
    t2_ref[...] = jnp.concatenate([v, -p, z], axis=1)


_init_call = pl.pallas_call(
    _init_body,
    grid=(NPAD // BN,),
    in_specs=[
        pl.BlockSpec((BN, F), lambda i: (i, 0)),
        pl.BlockSpec((BN, 3), lambda i: (i, 0)),
        pl.BlockSpec((F, H), lambda i: (0, 0)),
        pl.BlockSpec((1, H), lambda i: (0, 0)),
        pl.BlockSpec((2 * H + 7, H), lambda i: (0, 0)),
    ],
    out_specs=[
        pl.BlockSpec((BN, H), lambda i: (i, 0)),
        pl.BlockSpec((BN, TW), lambda i: (i, 0)),
        pl.BlockSpec((BN, TW), lambda i: (i, 0)),
    ],
    out_shape=[
        jax.ShapeDtypeStruct((NPAD, H), _F32),
        jax.ShapeDtypeStruct((NPAD, TW), _F32),
        jax.ShapeDtypeStruct((NPAD, TW), _F32),
    ],
)


def _edge_body(g_ref, ea_ref, ew1t_ref, eb1_ref, ew2_ref, eb2_ref,
               ew3_ref, eb3_ref, cw1_ref, cb1_ref, cw2_ref, cb2_ref,
               cw3_ref, cb3_ref, out_ref):
    g = g_ref[...]
    diff = g[:, H:H + 3]
    radial = jnp.sum(diff * diff, axis=1, keepdims=True)
    ew1t = ew1t_ref[...]
    pre = (g[:, :H]
           + radial * ew1t[0:1, :]
           + _dot(ea_ref[...], ew1t[1:5, :])
           + eb1_ref[...])
    h1 = _leaky(pre)
    h2 = _leaky(_dot(h1, ew2_ref[...]) + eb2_ref[...])
    ef = _dot(h2, ew3_ref[...]) + eb3_ref[...]
    c1 = _leaky(_dot(ef, cw1_ref[...]) + cb1_ref[...])
    c2 = _leaky(_dot(c1, cw2_ref[...]) + cb2_ref[...])
    cm = _dot(c2, cw3_ref[...]) + cb3_ref[...]
    trans = jnp.clip(diff * cm, -100.0, 100.0)
    ones = jnp.ones((BE, 1), _F32)
    zpad = jnp.zeros((BE, AW - 20), _F32)
    out_ref[...] = jnp.concatenate([ef, trans, ones, zpad], axis=1)


_edge_call = pl.pallas_call(
    _edge_body,
    grid=(E // BE,),
    in_specs=[
        pl.BlockSpec((BE, TW), lambda i: (i, 0)),
        pl.BlockSpec((BE, 4), lambda i: (i, 0)),
        pl.BlockSpec((5, H), lambda i: (0, 0)),
        pl.BlockSpec((1, H), lambda i: (0, 0)),
        pl.BlockSpec((H, H // 2), lambda i: (0, 0)),
        pl.BlockSpec((1, H // 2), lambda i: (0, 0)),
        pl.BlockSpec((H // 2, H // 4), lambda i: (0, 0)),
        pl.BlockSpec((1, H // 4), lambda i: (0, 0)),
        pl.BlockSpec((H // 4, H // 2), lambda i: (0, 0)),
        pl.BlockSpec((1, H // 2), lambda i: (0, 0)),
        pl.BlockSpec((H // 2, H // 2), lambda i: (0, 0)),
        pl.BlockSpec((1, H // 2), lambda i: (0, 0)),
        pl.BlockSpec((H // 2, 3), lambda i: (0, 0)),
        pl.BlockSpec((1, 3), lambda i: (0, 0)),
    ],
    out_specs=pl.BlockSpec((BE, AW), lambda i: (i, 0)),
    out_shape=jax.ShapeDtypeStruct((E, AW), _F32),
)


def _make_node_body(t_cur, t_next, dt):
    def body(h_ref, p_ref, aggA_ref, aggB_ref, nw1_ref, nb1_ref, nw2_ref,
             nb2_ref, nw3_ref, nb3_ref, ew1_ref,
             hn_ref, pn_ref, t1_ref, t2_ref):
        h = h_ref[...]
        p = p_ref[...]
        agg = aggA_ref[...] + aggB_ref[...]
        ef = agg[:, 0:16]
        sums = agg[:, 16:19]
        deg = agg[:, 19:20]
        nw1 = nw1_ref[...]
        o1 = _leaky(t_cur * nw1[0:1, :]
                    + _dot(h, nw1[1:H + 1, :])
                    + _dot(ef, nw1[H + 1:H + 17, :])
                    + nb1_ref[...])
        o2 = _leaky(_dot(o1, nw2_ref[...]) + nb2_ref[...])
        mlp = _dot(o2, nw3_ref[...]) + nb3_ref[...]
        hn = h + dt * mlp
        coord = p + sums / jnp.maximum(deg, 1.0)
        pn = p + dt * coord
        ew1 = ew1_ref[...]
        u = t_next * ew1[0:1, :] + _dot(hn, ew1[1:65, :])
        v = t_next * ew1[65:66, :] + _dot(hn, ew1[66:130, :])
        z = jnp.zeros((BN, TW - H - 3), _F32)
        hn_ref[...] = hn
        pn_ref[...] = pn
        t1_ref[...] = jnp.concatenate([u, pn, z], axis=1)
        t2_ref[...] = jnp.concatenate([v, -pn, z], axis=1)
    return body


def _make_node_call(t_cur, t_next, dt):
    return pl.pallas_call(
        _make_node_body(t_cur, t_next, dt),
        grid=(NPAD // BN,),
        in_specs=[
            pl.BlockSpec((BN, H), lambda i: (i, 0)),
            pl.BlockSpec((BN, 3), lambda i: (i, 0)),
            pl.BlockSpec((BN, AW), lambda i: (i, 0)),
            pl.BlockSpec((BN, AW), lambda i: (i + NPAD // BN, 0)),
            pl.BlockSpec((H + H // 4 + 1, 2 * H), lambda i: (0, 0)),
            pl.BlockSpec((1, 2 * H), lambda i: (0, 0)),
            pl.BlockSpec((2 * H, 2 * H), lambda i: (0, 0)),
            pl.BlockSpec((1, 2 * H), lambda i: (0, 0)),
            pl.BlockSpec((2 * H, H), lambda i: (0, 0)),
            pl.BlockSpec((1, H), lambda i: (0, 0)),
            pl.BlockSpec((2 * H + 7, H), lambda i: (0, 0)),
        ],
        out_specs=[
            pl.BlockSpec((BN, H), lambda i: (i, 0)),
            pl.BlockSpec((BN, 3), lambda i: (i, 0)),
            pl.BlockSpec((BN, TW), lambda i: (i, 0)),
            pl.BlockSpec((BN, TW), lambda i: (i, 0)),
        ],
        out_shape=[
            jax.ShapeDtypeStruct((NPAD, H), _F32),
            jax.ShapeDtypeStruct((NPAD, 3), _F32),
            jax.ShapeDtypeStruct((NPAD, TW), _F32),
            jax.ShapeDtypeStruct((NPAD, TW), _F32),
        ],
    )


_TS = np.linspace(0.0, 2.0, 4).astype(np.float32)
_node_calls = [
    _make_node_call(float(_TS[i]), float(_TS[i + 1]), float(_TS[i + 1] - _TS[i]))
    for i in range(3)
]


_MESH = plsc.VectorSubcoreMesh(core_axis_name="c", subcore_axis_name="s")


GC2 = 200
NCH = EPT // (2 * GC2)


def _add80(a, b):
    def inner(r, carry):
        for cidx in range(5):
            sl = pl.ds(cidx * 16, 16)
            plsc.addupdate(a.at[r, sl], b[r, sl])
        return carry
    lax.fori_loop(0, GC2, inner, 0)


@functools.partial(
    pl.kernel,
    out_type=jax.ShapeDtypeStruct((E, TW), _F32),
    mesh=_MESH,
    scratch_types=[
        pltpu.VMEM((GC2,), jnp.int32),
        pltpu.VMEM((GC2,), jnp.int32),
        pltpu.VMEM((GC2,), jnp.int32),
        pltpu.VMEM((GC2,), jnp.int32),
        pltpu.VMEM((GC2, TW), _F32),
        pltpu.VMEM((GC2, TW), _F32),
        pltpu.VMEM((GC2, TW), _F32),
        pltpu.VMEM((GC2, TW), _F32),
        pltpu.SemaphoreType.DMA,
        pltpu.SemaphoreType.DMA,
        pltpu.SemaphoreType.DMA,
        pltpu.SemaphoreType.DMA,
        pltpu.SemaphoreType.DMA,
        pltpu.SemaphoreType.DMA,
    ],
)
def _gather_call(t1_hbm, t2_hbm, row_hbm, col_hbm, g_out,
                 ra0, ca0, ra1, ca1, a0, b0, a1, b1,
                 sga0, sgb0, sga1, sgb1, sw0, sw1):
    wid = lax.axis_index("s") * 2 + lax.axis_index("c")
    base = wid * EPT

    def body(j, carry):
        offa = base + (2 * j) * GC2
        offb = offa + GC2
        pltpu.sync_copy(row_hbm.at[pl.ds(offa, GC2)], ra0)
        pltpu.sync_copy(col_hbm.at[pl.ds(offa, GC2)], ca0)
        pltpu.make_async_copy(t1_hbm.at[ra0], a0, sga0).start()
        pltpu.make_async_copy(t2_hbm.at[ca0], b0, sgb0).start()
        pltpu.sync_copy(row_hbm.at[pl.ds(offb, GC2)], ra1)
        pltpu.sync_copy(col_hbm.at[pl.ds(offb, GC2)], ca1)
        pltpu.make_async_copy(t1_hbm.at[ra1], a1, sga1).start()
        pltpu.make_async_copy(t2_hbm.at[ca1], b1, sgb1).start()
        pltpu.make_async_copy(t1_hbm.at[ra0], a0, sga0).wait()
        pltpu.make_async_copy(t2_hbm.at[ca0], b0, sgb0).wait()
        _add80(a0, b0)
        pltpu.make_async_copy(a0, g_out.at[pl.ds(offa, GC2)], sw0).start()
        pltpu.make_async_copy(t1_hbm.at[ra1], a1, sga1).wait()
        pltpu.make_async_copy(t2_hbm.at[ca1], b1, sgb1).wait()
        _add80(a1, b1)
        pltpu.make_async_copy(a1, g_out.at[pl.ds(offb, GC2)], sw1).start()
        pltpu.make_async_copy(a0, g_out.at[pl.ds(offa, GC2)], sw0).wait()
        pltpu.make_async_copy(a1, g_out.at[pl.ds(offb, GC2)], sw1).wait()
        return carry

    lax.fori_loop(0, NCH, body, 0)


@functools.partial(
    pl.kernel,
    out_type=jax.ShapeDtypeStruct((2 * NPAD, AW), _F32),
    mesh=_MESH,
    scratch_types=[
        pltpu.VMEM((GC,), jnp.int32),
        pltpu.VMEM((GC, AW), _F32),
        pltpu.VMEM((ROWS_PER_TILE, AW), _F32),
        pltpu.VMEM_SHARED((NPAD, AW), _F32),
    ],
    compiler_params=pltpu.CompilerParams(use_tc_tiling_on_sc=False),
)
def _scatter_call(efo_hbm, row_hbm, zeros_hbm, out_hbm, idx_v, buf, zbuf, acc):
    c = lax.axis_index("c")
    s = lax.axis_index("s")
    wid = s * 2 + c
    myrows = s * ROWS_PER_TILE
    pltpu.sync_copy(zeros_hbm.at[pl.ds(myrows, ROWS_PER_TILE)], zbuf)
    pltpu.sync_copy(zbuf, acc.at[pl.ds(myrows, ROWS_PER_TILE)])
    plsc.subcore_barrier()
    base = wid * EPT

    def body(i, carry):
        off = base + i * GC
        pltpu.sync_copy(row_hbm.at[pl.ds(off, GC)], idx_v)
        pltpu.sync_copy(efo_hbm.at[pl.ds(off, GC)], buf)
        pltpu.sync_copy(buf, acc.at[idx_v], add=True)
        return carry

    lax.fori_loop(0, EPT // GC, body, 0)
    plsc.subcore_barrier()
    pltpu.sync_copy(acc.at[pl.ds(myrows, ROWS_PER_TILE)], zbuf)
    pltpu.sync_copy(zbuf, out_hbm.at[pl.ds(c * NPAD + myrows, ROWS_PER_TILE)])


def kernel(x, pos, edge_attr, edge_index, W_emb, b_emb, ew1, eb1, ew2, eb2,
           ew3, eb3, cw1, cb1, cw2, cb2, cw3, cb3, nw1, nb1, nw2, nb2,
           nw3, nb3):
    row = edge_index[0]
    col = edge_index[1]
    xp = jnp.pad(x, ((0, NPAD - N), (0, 0)))
    pp = jnp.pad(pos, ((0, NPAD - N), (0, 0)))

    b_emb2 = b_emb.reshape(1, -1)
    eb1_2 = eb1.reshape(1, -1)
    eb2_2 = eb2.reshape(1, -1)
    eb3_2 = eb3.reshape(1, -1)
    cb1_2 = cb1.reshape(1, -1)
    cb2_2 = cb2.reshape(1, -1)
    cb3_2 = cb3.reshape(1, -1)
    nb1_2 = nb1.reshape(1, -1)
    nb2_2 = nb2.reshape(1, -1)
    nb3_2 = nb3.reshape(1, -1)
    ew1t = ew1[130:135]
    zeros_acc = jnp.zeros((NPAD, AW), _F32)

    h, t1, t2 = _init_call(xp, pp, W_emb, b_emb2, ew1)
    p = pp
    traj = [h]
    for i in range(3):
        g = _gather_call(t1, t2, row, col)
        efo = _edge_call(g, edge_attr, ew1t, eb1_2, ew2, eb2_2, ew3,
                         eb3_2, cw1, cb1_2, cw2, cb2_2, cw3, cb3_2)
        aggs = _scatter_call(efo, row, zeros_acc)
        h, p, t1, t2 = _node_calls[i](h, p, aggs, aggs, nw1, nb1_2, nw2,
                                      nb2_2, nw3, nb3_2, ew1)
        traj.append(h)
    return jnp.stack([hh[:N] for hh in traj], axis=0)


# speedup vs baseline: 6.9349x; 1.5863x over previous
import functools

import numpy as np
import jax
import jax.numpy as jnp
from jax import lax
from jax.experimental import pallas as pl
from jax.experimental.pallas import tpu as pltpu
from jax.experimental.pallas import tpu_sc as plsc

N = 10000
NPAD = 10240
E = 320000
F = 128
H = 64

TW = 128
AW = 32

NTILES = 32
EPT = E // NTILES
GC = 400
ROWS_PER_TILE = NPAD // 16

BN = 1024
BE = 4000

_F32 = jnp.float32


def _leaky(v):
    return jnp.where(v >= 0, v, 0.3 * v)


def _dot(a, b):
    return jnp.dot(a, b, preferred_element_type=_F32)


def _init_body(x_ref, pos_ref, wemb_ref, bemb_ref, ew1_ref,
               h_ref, t1_ref, t2_ref):
    h0 = _dot(x_ref[...], wemb_ref[...]) + bemb_ref[...]
    ew1 = ew1_ref[...]
    u = _dot(h0, ew1[1:65, :])
    v = _dot(h0, ew1[66:130, :])
    p = pos_ref[...]
    z = jnp.zeros((BN, TW - H - 3), _F32)
    h_ref[...] = h0
    t1_ref[...] = jnp.concatenate([u, p, z], axis=1)
    t2_ref[...] = jnp.concatenate([v, -p, z], axis=1)


_init_call = pl.pallas_call(
    _init_body,
    grid=(NPAD // BN,),
    in_specs=[
        pl.BlockSpec((BN, F), lambda i: (i, 0)),
        pl.BlockSpec((BN, 3), lambda i: (i, 0)),
        pl.BlockSpec((F, H), lambda i: (0, 0)),
        pl.BlockSpec((1, H), lambda i: (0, 0)),
        pl.BlockSpec((2 * H + 7, H), lambda i: (0, 0)),
    ],
    out_specs=[
        pl.BlockSpec((BN, H), lambda i: (i, 0)),
        pl.BlockSpec((BN, TW), lambda i: (i, 0)),
        pl.BlockSpec((BN, TW), lambda i: (i, 0)),
    ],
    out_shape=[
        jax.ShapeDtypeStruct((NPAD, H), _F32),
        jax.ShapeDtypeStruct((NPAD, TW), _F32),
        jax.ShapeDtypeStruct((NPAD, TW), _F32),
    ],
)


def _edge_body(g_ref, ea_ref, m128_ref, ew1t4_ref, eb1_ref, ew2_ref, eb2_ref,
               ew3e_ref, cw1e_ref, cb1x_ref, cw2_ref, cb2_ref,
               cw3e_ref, cb3e_ref, psel_ref, r32_ref, out_ref):
    g = g_ref[...]
    gsq = g * g
    pre = (g[:, :H]
           + _dot(gsq, m128_ref[...])
           + _dot(ea_ref[...], ew1t4_ref[...])
           + eb1_ref[...])
    h1 = _leaky(pre)
    h2 = _leaky(_dot(h1, ew2_ref[...]) + eb2_ref[...])
    z = _dot(h2, ew3e_ref[...])
    c1 = _leaky(_dot(z, cw1e_ref[...]) + cb1x_ref[...])
    c2 = _leaky(_dot(c1, cw2_ref[...]) + cb2_ref[...])
    cme = _dot(c2, cw3e_ref[...]) + cb3e_ref[...]
    diffe = _dot(g, psel_ref[...])
    trans_e = jnp.clip(diffe * cme, -100.0, 100.0)
    out_ref[...] = z + trans_e + r32_ref[...]


_edge_call = pl.pallas_call(
    _edge_body,
    grid=(E // BE,),
    in_specs=[
        pl.BlockSpec((BE, TW), lambda i: (i, 0)),
        pl.BlockSpec((BE, 4), lambda i: (i, 0)),
        pl.BlockSpec((TW, H), lambda i: (0, 0)),
        pl.BlockSpec((4, H), lambda i: (0, 0)),
        pl.BlockSpec((1, H), lambda i: (0, 0)),
        pl.BlockSpec((H, H // 2), lambda i: (0, 0)),
        pl.BlockSpec((1, H // 2), lambda i: (0, 0)),
        pl.BlockSpec((H // 2, AW), lambda i: (0, 0)),
        pl.BlockSpec((AW, H // 2), lambda i: (0, 0)),
        pl.BlockSpec((1, H // 2), lambda i: (0, 0)),
        pl.BlockSpec((H // 2, H // 2), lambda i: (0, 0)),
        pl.BlockSpec((1, H // 2), lambda i: (0, 0)),
        pl.BlockSpec((H // 2, AW), lambda i: (0, 0)),
        pl.BlockSpec((1, AW), lambda i: (0, 0)),
        pl.BlockSpec((TW, AW), lambda i: (0, 0)),
        pl.BlockSpec((1, AW), lambda i: (0, 0)),
    ],
    out_specs=pl.BlockSpec((BE, AW), lambda i: (i, 0)),
    out_shape=jax.ShapeDtypeStruct((E, AW), _F32),
)


def _make_node_body(t_cur, t_next, dt):
    def body(h_ref, p_ref, aggA_ref, aggB_ref, nw1_ref, nb1_ref, nw2_ref,
             nb2_ref, nw3_ref, nb3_ref, ew1_ref,
             hn_ref, pn_ref, t1_ref, t2_ref):
        h = h_ref[...]
        p = p_ref[...]
        agg = aggA_ref[...] + aggB_ref[...]
        ef = agg[:, 0:16]
        sums = agg[:, 16:19]
        deg = agg[:, 19:20]
        nw1 = nw1_ref[...]
        o1 = _leaky(t_cur * nw1[0:1, :]
                    + _dot(h, nw1[1:H + 1, :])
                    + _dot(ef, nw1[H + 1:H + 17, :])
                    + nb1_ref[...])
        o2 = _leaky(_dot(o1, nw2_ref[...]) + nb2_ref[...])
        mlp = _dot(o2, nw3_ref[...]) + nb3_ref[...]
        hn = h + dt * mlp
        coord = p + sums / jnp.maximum(deg, 1.0)
        pn = p + dt * coord
        ew1 = ew1_ref[...]
        u = t_next * ew1[0:1, :] + _dot(hn, ew1[1:65, :])
        v = t_next * ew1[65:66, :] + _dot(hn, ew1[66:130, :])
        z = jnp.zeros((BN, TW - H - 3), _F32)
        hn_ref[...] = hn
        pn_ref[...] = pn
        t1_ref[...] = jnp.concatenate([u, pn, z], axis=1)
        t2_ref[...] = jnp.concatenate([v, -pn, z], axis=1)
    return body


def _make_node_call(t_cur, t_next, dt):
    return pl.pallas_call(
        _make_node_body(t_cur, t_next, dt),
        grid=(NPAD // BN,),
        in_specs=[
            pl.BlockSpec((BN, H), lambda i: (i, 0)),
            pl.BlockSpec((BN, 3), lambda i: (i, 0)),
            pl.BlockSpec((BN, AW), lambda i: (i, 0)),
            pl.BlockSpec((BN, AW), lambda i: (i + NPAD // BN, 0)),
            pl.BlockSpec((H + H // 4 + 1, 2 * H), lambda i: (0, 0)),
            pl.BlockSpec((1, 2 * H), lambda i: (0, 0)),
            pl.BlockSpec((2 * H, 2 * H), lambda i: (0, 0)),
            pl.BlockSpec((1, 2 * H), lambda i: (0, 0)),
            pl.BlockSpec((2 * H, H), lambda i: (0, 0)),
            pl.BlockSpec((1, H), lambda i: (0, 0)),
            pl.BlockSpec((2 * H + 7, H), lambda i: (0, 0)),
        ],
        out_specs=[
            pl.BlockSpec((BN, H), lambda i: (i, 0)),
            pl.BlockSpec((BN, 3), lambda i: (i, 0)),
            pl.BlockSpec((BN, TW), lambda i: (i, 0)),
            pl.BlockSpec((BN, TW), lambda i: (i, 0)),
        ],
        out_shape=[
            jax.ShapeDtypeStruct((NPAD, H), _F32),
            jax.ShapeDtypeStruct((NPAD, 3), _F32),
            jax.ShapeDtypeStruct((NPAD, TW), _F32),
            jax.ShapeDtypeStruct((NPAD, TW), _F32),
        ],
    )


_TS = np.linspace(0.0, 2.0, 4).astype(np.float32)
_node_calls = [
    _make_node_call(float(_TS[i]), float(_TS[i + 1]), float(_TS[i + 1] - _TS[i]))
    for i in range(3)
]


_MESH = plsc.VectorSubcoreMesh(core_axis_name="c", subcore_axis_name="s")


GC2 = 200
NCH = EPT // (2 * GC2)


def _add80(a, b):
    def inner(r, carry):
        for cidx in range(5):
            sl = pl.ds(cidx * 16, 16)
            plsc.addupdate(a.at[r, sl], b[r, sl])
        return carry
    lax.fori_loop(0, GC2, inner, 0)


@functools.partial(
    pl.kernel,
    out_type=jax.ShapeDtypeStruct((E, TW), _F32),
    mesh=_MESH,
    scratch_types=[
        pltpu.VMEM((GC2,), jnp.int32),
        pltpu.VMEM((GC2,), jnp.int32),
        pltpu.VMEM((GC2,), jnp.int32),
        pltpu.VMEM((GC2,), jnp.int32),
        pltpu.VMEM((GC2, TW), _F32),
        pltpu.VMEM((GC2, TW), _F32),
        pltpu.VMEM((GC2, TW), _F32),
        pltpu.VMEM((GC2, TW), _F32),
        pltpu.SemaphoreType.DMA,
        pltpu.SemaphoreType.DMA,
        pltpu.SemaphoreType.DMA,
        pltpu.SemaphoreType.DMA,
        pltpu.SemaphoreType.DMA,
        pltpu.SemaphoreType.DMA,
    ],
)
def _gather_call(t1_hbm, t2_hbm, row_hbm, col_hbm, g_out,
                 ra0, ca0, ra1, ca1, a0, b0, a1, b1,
                 sga0, sgb0, sga1, sgb1, sw0, sw1):
    wid = lax.axis_index("s") * 2 + lax.axis_index("c")
    base = wid * EPT

    def body(j, carry):
        offa = base + (2 * j) * GC2
        offb = offa + GC2
        pltpu.sync_copy(row_hbm.at[pl.ds(offa, GC2)], ra0)
        pltpu.sync_copy(col_hbm.at[pl.ds(offa, GC2)], ca0)
        pltpu.make_async_copy(t1_hbm.at[ra0], a0, sga0).start()
        pltpu.make_async_copy(t2_hbm.at[ca0], b0, sgb0).start()
        pltpu.sync_copy(row_hbm.at[pl.ds(offb, GC2)], ra1)
        pltpu.sync_copy(col_hbm.at[pl.ds(offb, GC2)], ca1)
        pltpu.make_async_copy(t1_hbm.at[ra1], a1, sga1).start()
        pltpu.make_async_copy(t2_hbm.at[ca1], b1, sgb1).start()
        pltpu.make_async_copy(t1_hbm.at[ra0], a0, sga0).wait()
        pltpu.make_async_copy(t2_hbm.at[ca0], b0, sgb0).wait()
        _add80(a0, b0)
        pltpu.make_async_copy(a0, g_out.at[pl.ds(offa, GC2)], sw0).start()
        pltpu.make_async_copy(t1_hbm.at[ra1], a1, sga1).wait()
        pltpu.make_async_copy(t2_hbm.at[ca1], b1, sgb1).wait()
        _add80(a1, b1)
        pltpu.make_async_copy(a1, g_out.at[pl.ds(offb, GC2)], sw1).start()
        pltpu.make_async_copy(a0, g_out.at[pl.ds(offa, GC2)], sw0).wait()
        pltpu.make_async_copy(a1, g_out.at[pl.ds(offb, GC2)], sw1).wait()
        return carry

    lax.fori_loop(0, NCH, body, 0)


@functools.partial(
    pl.kernel,
    out_type=jax.ShapeDtypeStruct((2 * NPAD, AW), _F32),
    mesh=_MESH,
    scratch_types=[
        pltpu.VMEM((GC,), jnp.int32),
        pltpu.VMEM((GC, AW), _F32),
        pltpu.VMEM((ROWS_PER_TILE, AW), _F32),
        pltpu.VMEM_SHARED((NPAD, AW), _F32),
    ],
    compiler_params=pltpu.CompilerParams(use_tc_tiling_on_sc=False),
)
def _scatter_call(efo_hbm, row_hbm, zeros_hbm, out_hbm, idx_v, buf, zbuf, acc):
    c = lax.axis_index("c")
    s = lax.axis_index("s")
    wid = s * 2 + c
    myrows = s * ROWS_PER_TILE
    pltpu.sync_copy(zeros_hbm.at[pl.ds(myrows, ROWS_PER_TILE)], zbuf)
    pltpu.sync_copy(zbuf, acc.at[pl.ds(myrows, ROWS_PER_TILE)])
    plsc.subcore_barrier()
    base = wid * EPT

    def body(i, carry):
        off = base + i * GC
        pltpu.sync_copy(row_hbm.at[pl.ds(off, GC)], idx_v)
        pltpu.sync_copy(efo_hbm.at[pl.ds(off, GC)], buf)
        pltpu.sync_copy(buf, acc.at[idx_v], add=True)
        return carry

    lax.fori_loop(0, EPT // GC, body, 0)
    plsc.subcore_barrier()
    pltpu.sync_copy(acc.at[pl.ds(myrows, ROWS_PER_TILE)], zbuf)
    pltpu.sync_copy(zbuf, out_hbm.at[pl.ds(c * NPAD + myrows, ROWS_PER_TILE)])


def kernel(x, pos, edge_attr, edge_index, W_emb, b_emb, ew1, eb1, ew2, eb2,
           ew3, eb3, cw1, cb1, cw2, cb2, cw3, cb3, nw1, nb1, nw2, nb2,
           nw3, nb3):
    row = edge_index[0]
    col = edge_index[1]
    xp = jnp.pad(x, ((0, NPAD - N), (0, 0)))
    pp = jnp.pad(pos, ((0, NPAD - N), (0, 0)))

    b_emb2 = b_emb.reshape(1, -1)
    eb1_2 = eb1.reshape(1, -1)
    eb2_2 = eb2.reshape(1, -1)
    eb3_2 = eb3.reshape(1, -1)
    cb1_2 = cb1.reshape(1, -1)
    cb2_2 = cb2.reshape(1, -1)
    cb3_2 = cb3.reshape(1, -1)
    nb1_2 = nb1.reshape(1, -1)
    nb2_2 = nb2.reshape(1, -1)
    nb3_2 = nb3.reshape(1, -1)
    m128 = jnp.zeros((TW, H), _F32).at[H:H + 3, :].set(
        jnp.broadcast_to(ew1[130:131, :], (3, H)))
    ew1t4 = ew1[131:135]
    ew3e = jnp.zeros((H // 2, AW), _F32).at[:, 0:16].set(ew3)
    cw1e = jnp.zeros((AW, H // 2), _F32).at[0:16, :].set(cw1)
    cb1x = (eb3.reshape(1, -1) @ cw1 + cb1.reshape(1, -1))
    cw3e = jnp.zeros((H // 2, AW), _F32).at[:, 16:19].set(cw3)
    cb3e = jnp.zeros((1, AW), _F32).at[0, 16:19].set(cb3)
    psel = jnp.zeros((TW, AW), _F32).at[H, 16].set(1.0).at[H + 1, 17].set(1.0).at[H + 2, 18].set(1.0)
    r32 = jnp.zeros((1, AW), _F32).at[0, 0:16].set(eb3).at[0, 19].set(1.0)
    zeros_acc = jnp.zeros((NPAD, AW), _F32)

    h, t1, t2 = _init_call(xp, pp, W_emb, b_emb2, ew1)
    p = pp
    traj = [h]
    for i in range(3):
        g = _gather_call(t1, t2, row, col)
        efo = _edge_call(g, edge_attr, m128, ew1t4, eb1_2, ew2, eb2_2,
                         ew3e, cw1e, cb1x, cw2, cb2_2, cw3e, cb3e, psel, r32)
        aggs = _scatter_call(efo, row, zeros_acc)
        h, p, t1, t2 = _node_calls[i](h, p, aggs, aggs, nw1, nb1_2, nw2,
                                      nb2_2, nw3, nb3_2, ew1)
        traj.append(h)
    return jnp.stack([hh[:N] for hh in traj], axis=0)
